# baseline (speedup 1.0000x reference)
import functools

import jax
import jax.numpy as jnp
from jax.experimental import pallas as pl
from jax.experimental.pallas import tpu as pltpu

N = 4096
D = 256
ATT = 128
SEG = 4
BM = 512

_PREC = jax.lax.Precision.HIGHEST


def _dot(a, b, dims):
    return jax.lax.dot_general(a, b, dimension_numbers=(dims, ((), ())),
                               preferred_element_type=jnp.float32,
                               precision=_PREC)


def _gram_block(xb, x):
    l2 = jnp.sum(x * x, axis=1, keepdims=True)
    l2b = jnp.sum(xb * xb, axis=1, keepdims=True)
    dot = _dot(xb, x, ((1,), (1,)))
    return l2b + l2.reshape(1, -1) - 2.0 * dot


def _attention_pool(h, Wa, ba, Wb, bb):
    t = jnp.tanh(_dot(h, Wa, ((1,), (0,))) + ba.reshape(1, -1))
    g = _dot(t, Wb, ((1,), (0,))) + bb.reshape(1, 1)
    row = jax.lax.broadcasted_iota(jnp.int32, (N, SEG), 0)
    col = jax.lax.broadcasted_iota(jnp.int32, (N, SEG), 1)
    mask = (row % SEG) == col
    ninf = jnp.float32(-jnp.inf)
    gm = jnp.max(jnp.where(mask, g, ninf), axis=0, keepdims=True)
    e = jnp.where(mask, jnp.exp(g - gm), 0.0)
    denom = jnp.sum(e, axis=0, keepdims=True)
    a = e / denom
    return _dot(a, h, ((0,), (0,)))


def _diag(row):
    p = jax.lax.broadcasted_iota(jnp.int32, (SEG, SEG), 0)
    q = jax.lax.broadcasted_iota(jnp.int32, (SEG, SEG), 1)
    return jnp.where(p == q, row, 0.0)


def _mega_kernel(xb_ref, x_ref, W1_ref, b1_ref, W2_ref, b2_ref, W3_ref,
                 b3_ref, Wa_ref, ba_ref, Wb_ref, bb_ref,
                 mn_ref, out_ref, m_ref, den_ref, l2col_ref, xm2_ref):
    i = pl.program_id(0)
    nsteps = pl.num_programs(0)
    xb = xb_ref[...]

    @pl.when(i == 0)
    def _():
        x = x_ref[...]
        l2col_ref[...] = jnp.sum(x * x, axis=1, keepdims=True)
        xm2_ref[...] = (-2.0 * x).astype(jnp.bfloat16)

    l2b_row = _dot(jnp.ones((1, D), jnp.float32), xb * xb, ((1,), (1,)))
    dm2t = jax.lax.dot_general(
        xm2_ref[...], xb.astype(jnp.bfloat16),
        dimension_numbers=((((1,), (1,))), ((), ())),
        preferred_element_type=jnp.float32)
    u = l2col_ref[...] + dm2t
    colmin = jnp.min(u, axis=0, keepdims=True)
    mv = jnp.min(colmin + l2b_row).reshape(1, 1)

    @pl.when(i == 0)
    def _():
        mn_ref[...] = mv

    @pl.when(i != 0)
    def _():
        mn_ref[...] = jnp.minimum(mn_ref[...], mv)

    h = xb
    for W_ref, b_ref in ((W1_ref, b1_ref), (W2_ref, b2_ref), (W3_ref, b3_ref)):
        Wd = W_ref[0] - W_ref[2]
        h = jnp.maximum(_dot(h, Wd, ((1,), (0,))) + b_ref[...].reshape(1, -1), 0.0)
    t = jnp.tanh(_dot(h, Wa_ref[...], ((1,), (0,))) + ba_ref[...].reshape(1, -1))
    g = _dot(t, Wb_ref[...], ((1,), (0,))) + bb_ref[...].reshape(1, 1)

    row = jax.lax.broadcasted_iota(jnp.int32, (BM, SEG), 0)
    col = jax.lax.broadcasted_iota(jnp.int32, (BM, SEG), 1)
    mask = (row % SEG) == col
    ninf = jnp.float32(-jnp.inf)
    mb = jnp.max(jnp.where(mask, g, ninf), axis=0, keepdims=True)

    @pl.when(i == 0)
    def _():
        e = jnp.where(mask, jnp.exp(g - mb), 0.0)
        m_ref[...] = mb
        den_ref[...] = jnp.sum(e, axis=0, keepdims=True)
        out_ref[...] = _dot(e, h, ((0,), (0,)))

    @pl.when(i != 0)
    def _():
        m_old = m_ref[...]
        m_new = jnp.maximum(m_old, mb)
        scale = jnp.exp(m_old - m_new)
        e = jnp.where(mask, jnp.exp(g - m_new), 0.0)
        den_ref[...] = den_ref[...] * scale + jnp.sum(e, axis=0, keepdims=True)
        out_ref[...] = (_dot(_diag(scale), out_ref[...], ((1,), (0,)))
                        + _dot(e, h, ((0,), (0,))))
        m_ref[...] = m_new

    @pl.when(i == nsteps - 1)
    def _():
        out_ref[...] = _dot(_diag(1.0 / den_ref[...]), out_ref[...],
                            ((1,), (0,)))


def _mega(x, W1, b1, W2, b2, W3, b3, Wa, ba, Wb, bb):
    mn, out = pl.pallas_call(
        _mega_kernel,
        grid=(N // BM,),
        in_specs=[
            pl.BlockSpec((BM, D), lambda i: (i, 0)),
            pl.BlockSpec((N, D), lambda i: (0, 0)),
            pl.BlockSpec((3, D, D), lambda i: (0, 0, 0)),
            pl.BlockSpec((D,), lambda i: (0,)),
            pl.BlockSpec((3, D, D), lambda i: (0, 0, 0)),
            pl.BlockSpec((D,), lambda i: (0,)),
            pl.BlockSpec((3, D, D), lambda i: (0, 0, 0)),
            pl.BlockSpec((D,), lambda i: (0,)),
            pl.BlockSpec((D, ATT), lambda i: (0, 0)),
            pl.BlockSpec((ATT,), lambda i: (0,)),
            pl.BlockSpec((ATT, 1), lambda i: (0, 0)),
            pl.BlockSpec((1,), lambda i: (0,)),
        ],
        out_specs=[
            pl.BlockSpec((1, 1), lambda i: (0, 0)),
            pl.BlockSpec((SEG, D), lambda i: (0, 0)),
        ],
        out_shape=[
            jax.ShapeDtypeStruct((1, 1), jnp.float32),
            jax.ShapeDtypeStruct((SEG, D), jnp.float32),
        ],
        scratch_shapes=[
            pltpu.VMEM((1, SEG), jnp.float32),
            pltpu.VMEM((1, SEG), jnp.float32),
            pltpu.VMEM((N, 1), jnp.float32),
            pltpu.VMEM((N, D), jnp.bfloat16),
        ],
    )(x, x, W1, b1, W2, b2, W3, b3, Wa, ba, Wb, bb)
    return mn[0, 0], out


def _maxv_kernel(xb_ref, x_ref, maxv_ref):
    i = pl.program_id(0)
    m = jnp.max(_gram_block(xb_ref[...], x_ref[...])).reshape(1, 1)

    @pl.when(i == 0)
    def _():
        maxv_ref[...] = m

    @pl.when(i != 0)
    def _():
        maxv_ref[...] = jnp.maximum(maxv_ref[...], m)

def _adj_block(xb, x, thresh, i):
    v = _gram_block(xb, x)
    dist = jnp.sqrt(v)
    adj = dist < thresh
    gi = jax.lax.broadcasted_iota(jnp.int32, (BM, N), 0) + i * BM
    gj = jax.lax.broadcasted_iota(jnp.int32, (BM, N), 1)
    same = (gi % SEG) == (gj % SEG)
    return adj & jnp.logical_not(same)


def _deg_kernel(thresh_ref, xb_ref, x_ref, deg_ref):
    i = pl.program_id(0)
    adj = _adj_block(xb_ref[...], x_ref[...], thresh_ref[...], i)
    deg_ref[...] = jnp.sum(jnp.where(adj, 1.0, 0.0), axis=1, keepdims=True)


def _lw_kernel(thresh_ref, xb_ref, x_ref, deg_ref, lw_ref):
    i = pl.program_id(0)
    adj = _adj_block(xb_ref[...], x_ref[...], thresh_ref[...], i)
    deg = deg_ref[...]
    dinv = deg ** -0.5
    dinv = jnp.where(jnp.isinf(dinv), 0.0, dinv)
    deg_b = deg_ref[pl.ds(i * BM, BM), :]
    dinv_b = deg_b ** -0.5
    dinv_b = jnp.where(jnp.isinf(dinv_b), 0.0, dinv_b)
    lw_ref[...] = jnp.where(adj, (-dinv_b) * dinv.reshape(1, -1), 0.0)


def _lap_kernel(lw_ref, h_ref, out_ref):
    out_ref[...] = _dot(lw_ref[...], h_ref[...], ((1,), (0,)))


def _cheb_out_kernel(lw_ref, tx1_ref, h_ref, W_ref, b_ref, out_ref):
    i = pl.program_id(0)
    hb = h_ref[pl.ds(i * BM, BM), :]
    tx1b = tx1_ref[pl.ds(i * BM, BM), :]
    tx2b = 2.0 * _dot(lw_ref[...], tx1_ref[...], ((1,), (0,))) - hb
    acc = (_dot(hb, W_ref[0], ((1,), (0,)))
           + _dot(tx1b, W_ref[1], ((1,), (0,)))
           + _dot(tx2b, W_ref[2], ((1,), (0,)))
           + b_ref[...].reshape(1, -1))
    out_ref[...] = jnp.maximum(acc, 0.0)


def _pool_kernel(h_ref, Wa_ref, ba_ref, Wb_ref, bb_ref, out_ref):
    out_ref[...] = _attention_pool(h_ref[...], Wa_ref[...], ba_ref[...],
                                   Wb_ref[...], bb_ref[...])


def _full_graph_path(x, W1, b1, W2, b2, W3, b3, Wa, ba, Wb, bb):
    maxv = pl.pallas_call(
        _maxv_kernel,
        grid=(N // BM,),
        in_specs=[
            pl.BlockSpec((BM, D), lambda i: (i, 0)),
            pl.BlockSpec((N, D), lambda i: (0, 0)),
        ],
        out_specs=pl.BlockSpec((1, 1), lambda i: (0, 0)),
        out_shape=jax.ShapeDtypeStruct((1, 1), jnp.float32),
    )(x, x)
    thresh = 0.5 * jnp.sqrt(maxv)

    deg = pl.pallas_call(
        _deg_kernel,
        grid=(N // BM,),
        in_specs=[
            pl.BlockSpec((1, 1), lambda i: (0, 0)),
            pl.BlockSpec((BM, D), lambda i: (i, 0)),
            pl.BlockSpec((N, D), lambda i: (0, 0)),
        ],
        out_specs=pl.BlockSpec((BM, 1), lambda i: (i, 0)),
        out_shape=jax.ShapeDtypeStruct((N, 1), jnp.float32),
    )(thresh, x, x)

    lw = pl.pallas_call(
        _lw_kernel,
        grid=(N // BM,),
        in_specs=[
            pl.BlockSpec((1, 1), lambda i: (0, 0)),
            pl.BlockSpec((BM, D), lambda i: (i, 0)),
            pl.BlockSpec((N, D), lambda i: (0, 0)),
            pl.BlockSpec((N, 1), lambda i: (0, 0)),
        ],
        out_specs=pl.BlockSpec((BM, N), lambda i: (i, 0)),
        out_shape=jax.ShapeDtypeStruct((N, N), jnp.float32),
    )(thresh, x, x, deg)

    lap = pl.pallas_call(
        _lap_kernel,
        grid=(N // BM,),
        in_specs=[
            pl.BlockSpec((BM, N), lambda i: (i, 0)),
            pl.BlockSpec((N, D), lambda i: (0, 0)),
        ],
        out_specs=pl.BlockSpec((BM, D), lambda i: (i, 0)),
        out_shape=jax.ShapeDtypeStruct((N, D), jnp.float32),
    )

    cheb_out = pl.pallas_call(
        _cheb_out_kernel,
        grid=(N // BM,),
        in_specs=[
            pl.BlockSpec((BM, N), lambda i: (i, 0)),
            pl.BlockSpec((N, D), lambda i: (0, 0)),
            pl.BlockSpec((N, D), lambda i: (0, 0)),
            pl.BlockSpec((3, D, D), lambda i: (0, 0, 0)),
            pl.BlockSpec((D,), lambda i: (0,)),
        ],
        out_specs=pl.BlockSpec((BM, D), lambda i: (i, 0)),
        out_shape=jax.ShapeDtypeStruct((N, D), jnp.float32),
    )

    h = x
    for W, b in ((W1, b1), (W2, b2), (W3, b3)):
        tx1 = lap(lw, h)
        h = cheb_out(lw, tx1, h, W, b)

    return pl.pallas_call(
        _pool_kernel,
        out_shape=jax.ShapeDtypeStruct((SEG, D), jnp.float32),
    )(h, Wa, ba, Wb, bb)


def kernel(feats, W1, b1, W2, b2, W3, b3, Wa, ba, Wb, bb):
    bs, bag, d = feats.shape
    x = feats.reshape(bs * bag, d)
    mn, out_fast = _mega(x, W1, b1, W2, b2, W3, b3, Wa, ba, Wb, bb)
    out = jax.lax.cond(
        mn < 0,
        lambda ops: ops[0],
        lambda ops: _full_graph_path(*ops[1:]),
        (out_fast, x, W1, b1, W2, b2, W3, b3, Wa, ba, Wb, bb),
    )
    return out.reshape(bs, d)


# speedup vs baseline: 1.0969x; 1.0969x over previous
import functools

import jax
import jax.numpy as jnp
from jax.experimental import pallas as pl
from jax.experimental.pallas import tpu as pltpu

N = 4096
D = 256
ATT = 128
SEG = 4
BM = 1024

_PREC = jax.lax.Precision.HIGHEST


def _dot(a, b, dims):
    return jax.lax.dot_general(a, b, dimension_numbers=(dims, ((), ())),
                               preferred_element_type=jnp.float32,
                               precision=_PREC)


def _gram_block(xb, x):
    l2 = jnp.sum(x * x, axis=1, keepdims=True)
    l2b = jnp.sum(xb * xb, axis=1, keepdims=True)
    dot = _dot(xb, x, ((1,), (1,)))
    return l2b + l2.reshape(1, -1) - 2.0 * dot


def _attention_pool(h, Wa, ba, Wb, bb):
    t = jnp.tanh(_dot(h, Wa, ((1,), (0,))) + ba.reshape(1, -1))
    g = _dot(t, Wb, ((1,), (0,))) + bb.reshape(1, 1)
    row = jax.lax.broadcasted_iota(jnp.int32, (N, SEG), 0)
    col = jax.lax.broadcasted_iota(jnp.int32, (N, SEG), 1)
    mask = (row % SEG) == col
    ninf = jnp.float32(-jnp.inf)
    gm = jnp.max(jnp.where(mask, g, ninf), axis=0, keepdims=True)
    e = jnp.where(mask, jnp.exp(g - gm), 0.0)
    denom = jnp.sum(e, axis=0, keepdims=True)
    a = e / denom
    return _dot(a, h, ((0,), (0,)))


def _diag(row):
    p = jax.lax.broadcasted_iota(jnp.int32, (SEG, SEG), 0)
    q = jax.lax.broadcasted_iota(jnp.int32, (SEG, SEG), 1)
    return jnp.where(p == q, row, 0.0)


def _mega_kernel(xb_ref, x_ref, W1_ref, b1_ref, W2_ref, b2_ref, W3_ref,
                 b3_ref, Wa_ref, ba_ref, Wb_ref, bb_ref,
                 mn_ref, out_ref, m_ref, den_ref, l2col_ref, xm2_ref):
    i = pl.program_id(0)
    nsteps = pl.num_programs(0)
    xb = xb_ref[...]

    @pl.when(i == 0)
    def _():
        x = x_ref[...]
        l2col_ref[...] = jnp.sum(x * x, axis=1, keepdims=True)
        xm2_ref[...] = (-2.0 * x).astype(jnp.bfloat16)

    l2b_row = _dot(jnp.ones((1, D), jnp.float32), xb * xb, ((1,), (1,)))
    dm2t = jax.lax.dot_general(
        xm2_ref[...], xb.astype(jnp.bfloat16),
        dimension_numbers=((((1,), (1,))), ((), ())),
        preferred_element_type=jnp.float32)
    u = l2col_ref[...] + dm2t
    colmin = jnp.min(u, axis=0, keepdims=True)
    mv = jnp.min(colmin + l2b_row).reshape(1, 1)

    @pl.when(i == 0)
    def _():
        mn_ref[...] = mv

    @pl.when(i != 0)
    def _():
        mn_ref[...] = jnp.minimum(mn_ref[...], mv)

    h = xb
    for W_ref, b_ref in ((W1_ref, b1_ref), (W2_ref, b2_ref), (W3_ref, b3_ref)):
        Wd = W_ref[0] - W_ref[2]
        h = jnp.maximum(_dot(h, Wd, ((1,), (0,))) + b_ref[...].reshape(1, -1), 0.0)
    t = jnp.tanh(_dot(h, Wa_ref[...], ((1,), (0,))) + ba_ref[...].reshape(1, -1))
    g = _dot(t, Wb_ref[...], ((1,), (0,))) + bb_ref[...].reshape(1, 1)

    row = jax.lax.broadcasted_iota(jnp.int32, (BM, SEG), 0)
    col = jax.lax.broadcasted_iota(jnp.int32, (BM, SEG), 1)
    mask = (row % SEG) == col
    ninf = jnp.float32(-jnp.inf)
    mb = jnp.max(jnp.where(mask, g, ninf), axis=0, keepdims=True)

    @pl.when(i == 0)
    def _():
        e = jnp.where(mask, jnp.exp(g - mb), 0.0)
        m_ref[...] = mb
        den_ref[...] = jnp.sum(e, axis=0, keepdims=True)
        out_ref[...] = _dot(e, h, ((0,), (0,)))

    @pl.when(i != 0)
    def _():
        m_old = m_ref[...]
        m_new = jnp.maximum(m_old, mb)
        scale = jnp.exp(m_old - m_new)
        e = jnp.where(mask, jnp.exp(g - m_new), 0.0)
        den_ref[...] = den_ref[...] * scale + jnp.sum(e, axis=0, keepdims=True)
        out_ref[...] = (_dot(_diag(scale), out_ref[...], ((1,), (0,)))
                        + _dot(e, h, ((0,), (0,))))
        m_ref[...] = m_new

    @pl.when(i == nsteps - 1)
    def _():
        out_ref[...] = _dot(_diag(1.0 / den_ref[...]), out_ref[...],
                            ((1,), (0,)))


def _mega(x, W1, b1, W2, b2, W3, b3, Wa, ba, Wb, bb):
    mn, out = pl.pallas_call(
        _mega_kernel,
        grid=(N // BM,),
        in_specs=[
            pl.BlockSpec((BM, D), lambda i: (i, 0)),
            pl.BlockSpec((N, D), lambda i: (0, 0)),
            pl.BlockSpec((3, D, D), lambda i: (0, 0, 0)),
            pl.BlockSpec((D,), lambda i: (0,)),
            pl.BlockSpec((3, D, D), lambda i: (0, 0, 0)),
            pl.BlockSpec((D,), lambda i: (0,)),
            pl.BlockSpec((3, D, D), lambda i: (0, 0, 0)),
            pl.BlockSpec((D,), lambda i: (0,)),
            pl.BlockSpec((D, ATT), lambda i: (0, 0)),
            pl.BlockSpec((ATT,), lambda i: (0,)),
            pl.BlockSpec((ATT, 1), lambda i: (0, 0)),
            pl.BlockSpec((1,), lambda i: (0,)),
        ],
        out_specs=[
            pl.BlockSpec((1, 1), lambda i: (0, 0)),
            pl.BlockSpec((SEG, D), lambda i: (0, 0)),
        ],
        out_shape=[
            jax.ShapeDtypeStruct((1, 1), jnp.float32),
            jax.ShapeDtypeStruct((SEG, D), jnp.float32),
        ],
        scratch_shapes=[
            pltpu.VMEM((1, SEG), jnp.float32),
            pltpu.VMEM((1, SEG), jnp.float32),
            pltpu.VMEM((N, 1), jnp.float32),
            pltpu.VMEM((N, D), jnp.bfloat16),
        ],
    )(x, x, W1, b1, W2, b2, W3, b3, Wa, ba, Wb, bb)
    return mn[0, 0], out


def _maxv_kernel(xb_ref, x_ref, maxv_ref):
    i = pl.program_id(0)
    m = jnp.max(_gram_block(xb_ref[...], x_ref[...])).reshape(1, 1)

    @pl.when(i == 0)
    def _():
        maxv_ref[...] = m

    @pl.when(i != 0)
    def _():
        maxv_ref[...] = jnp.maximum(maxv_ref[...], m)

def _adj_block(xb, x, thresh, i):
    v = _gram_block(xb, x)
    dist = jnp.sqrt(v)
    adj = dist < thresh
    gi = jax.lax.broadcasted_iota(jnp.int32, (BM, N), 0) + i * BM
    gj = jax.lax.broadcasted_iota(jnp.int32, (BM, N), 1)
    same = (gi % SEG) == (gj % SEG)
    return adj & jnp.logical_not(same)


def _deg_kernel(thresh_ref, xb_ref, x_ref, deg_ref):
    i = pl.program_id(0)
    adj = _adj_block(xb_ref[...], x_ref[...], thresh_ref[...], i)
    deg_ref[...] = jnp.sum(jnp.where(adj, 1.0, 0.0), axis=1, keepdims=True)


def _lw_kernel(thresh_ref, xb_ref, x_ref, deg_ref, lw_ref):
    i = pl.program_id(0)
    adj = _adj_block(xb_ref[...], x_ref[...], thresh_ref[...], i)
    deg = deg_ref[...]
    dinv = deg ** -0.5
    dinv = jnp.where(jnp.isinf(dinv), 0.0, dinv)
    deg_b = deg_ref[pl.ds(i * BM, BM), :]
    dinv_b = deg_b ** -0.5
    dinv_b = jnp.where(jnp.isinf(dinv_b), 0.0, dinv_b)
    lw_ref[...] = jnp.where(adj, (-dinv_b) * dinv.reshape(1, -1), 0.0)


def _lap_kernel(lw_ref, h_ref, out_ref):
    out_ref[...] = _dot(lw_ref[...], h_ref[...], ((1,), (0,)))


def _cheb_out_kernel(lw_ref, tx1_ref, h_ref, W_ref, b_ref, out_ref):
    i = pl.program_id(0)
    hb = h_ref[pl.ds(i * BM, BM), :]
    tx1b = tx1_ref[pl.ds(i * BM, BM), :]
    tx2b = 2.0 * _dot(lw_ref[...], tx1_ref[...], ((1,), (0,))) - hb
    acc = (_dot(hb, W_ref[0], ((1,), (0,)))
           + _dot(tx1b, W_ref[1], ((1,), (0,)))
           + _dot(tx2b, W_ref[2], ((1,), (0,)))
           + b_ref[...].reshape(1, -1))
    out_ref[...] = jnp.maximum(acc, 0.0)


def _pool_kernel(h_ref, Wa_ref, ba_ref, Wb_ref, bb_ref, out_ref):
    out_ref[...] = _attention_pool(h_ref[...], Wa_ref[...], ba_ref[...],
                                   Wb_ref[...], bb_ref[...])


def _full_graph_path(x, W1, b1, W2, b2, W3, b3, Wa, ba, Wb, bb):
    maxv = pl.pallas_call(
        _maxv_kernel,
        grid=(N // BM,),
        in_specs=[
            pl.BlockSpec((BM, D), lambda i: (i, 0)),
            pl.BlockSpec((N, D), lambda i: (0, 0)),
        ],
        out_specs=pl.BlockSpec((1, 1), lambda i: (0, 0)),
        out_shape=jax.ShapeDtypeStruct((1, 1), jnp.float32),
    )(x, x)
    thresh = 0.5 * jnp.sqrt(maxv)

    deg = pl.pallas_call(
        _deg_kernel,
        grid=(N // BM,),
        in_specs=[
            pl.BlockSpec((1, 1), lambda i: (0, 0)),
            pl.BlockSpec((BM, D), lambda i: (i, 0)),
            pl.BlockSpec((N, D), lambda i: (0, 0)),
        ],
        out_specs=pl.BlockSpec((BM, 1), lambda i: (i, 0)),
        out_shape=jax.ShapeDtypeStruct((N, 1), jnp.float32),
    )(thresh, x, x)

    lw = pl.pallas_call(
        _lw_kernel,
        grid=(N // BM,),
        in_specs=[
            pl.BlockSpec((1, 1), lambda i: (0, 0)),
            pl.BlockSpec((BM, D), lambda i: (i, 0)),
            pl.BlockSpec((N, D), lambda i: (0, 0)),
            pl.BlockSpec((N, 1), lambda i: (0, 0)),
        ],
        out_specs=pl.BlockSpec((BM, N), lambda i: (i, 0)),
        out_shape=jax.ShapeDtypeStruct((N, N), jnp.float32),
    )(thresh, x, x, deg)

    lap = pl.pallas_call(
        _lap_kernel,
        grid=(N // BM,),
        in_specs=[
            pl.BlockSpec((BM, N), lambda i: (i, 0)),
            pl.BlockSpec((N, D), lambda i: (0, 0)),
        ],
        out_specs=pl.BlockSpec((BM, D), lambda i: (i, 0)),
        out_shape=jax.ShapeDtypeStruct((N, D), jnp.float32),
    )

    cheb_out = pl.pallas_call(
        _cheb_out_kernel,
        grid=(N // BM,),
        in_specs=[
            pl.BlockSpec((BM, N), lambda i: (i, 0)),
            pl.BlockSpec((N, D), lambda i: (0, 0)),
            pl.BlockSpec((N, D), lambda i: (0, 0)),
            pl.BlockSpec((3, D, D), lambda i: (0, 0, 0)),
            pl.BlockSpec((D,), lambda i: (0,)),
        ],
        out_specs=pl.BlockSpec((BM, D), lambda i: (i, 0)),
        out_shape=jax.ShapeDtypeStruct((N, D), jnp.float32),
    )

    h = x
    for W, b in ((W1, b1), (W2, b2), (W3, b3)):
        tx1 = lap(lw, h)
        h = cheb_out(lw, tx1, h, W, b)

    return pl.pallas_call(
        _pool_kernel,
        out_shape=jax.ShapeDtypeStruct((SEG, D), jnp.float32),
    )(h, Wa, ba, Wb, bb)


def kernel(feats, W1, b1, W2, b2, W3, b3, Wa, ba, Wb, bb):
    bs, bag, d = feats.shape
    x = feats.reshape(bs * bag, d)
    mn, out_fast = _mega(x, W1, b1, W2, b2, W3, b3, Wa, ba, Wb, bb)
    out = jax.lax.cond(
        mn < 0,
        lambda ops: ops[0],
        lambda ops: _full_graph_path(*ops[1:]),
        (out_fast, x, W1, b1, W2, b2, W3, b3, Wa, ba, Wb, bb),
    )
    return out.reshape(bs, d)


# speedup vs baseline: 2.3718x; 2.1624x over previous
import functools

import jax
import jax.numpy as jnp
from jax.experimental import pallas as pl
from jax.experimental.pallas import tpu as pltpu

N = 4096
D = 256
ATT = 128
SEG = 4
BM = 1024

_PREC = jax.lax.Precision.DEFAULT


def _dot(a, b, dims):
    return jax.lax.dot_general(a, b, dimension_numbers=(dims, ((), ())),
                               preferred_element_type=jnp.float32,
                               precision=_PREC)


def _gram_block(xb, x):
    l2 = jnp.sum(x * x, axis=1, keepdims=True)
    l2b = jnp.sum(xb * xb, axis=1, keepdims=True)
    dot = _dot(xb, x, ((1,), (1,)))
    return l2b + l2.reshape(1, -1) - 2.0 * dot


def _attention_pool(h, Wa, ba, Wb, bb):
    t = jnp.tanh(_dot(h, Wa, ((1,), (0,))) + ba.reshape(1, -1))
    g = _dot(t, Wb, ((1,), (0,))) + bb.reshape(1, 1)
    row = jax.lax.broadcasted_iota(jnp.int32, (N, SEG), 0)
    col = jax.lax.broadcasted_iota(jnp.int32, (N, SEG), 1)
    mask = (row % SEG) == col
    ninf = jnp.float32(-jnp.inf)
    gm = jnp.max(jnp.where(mask, g, ninf), axis=0, keepdims=True)
    e = jnp.where(mask, jnp.exp(g - gm), 0.0)
    denom = jnp.sum(e, axis=0, keepdims=True)
    a = e / denom
    return _dot(a, h, ((0,), (0,)))


def _diag(row):
    p = jax.lax.broadcasted_iota(jnp.int32, (SEG, SEG), 0)
    q = jax.lax.broadcasted_iota(jnp.int32, (SEG, SEG), 1)
    return jnp.where(p == q, row, 0.0)


def _mega_kernel(xb_ref, x_ref, W1_ref, b1_ref, W2_ref, b2_ref, W3_ref,
                 b3_ref, Wa_ref, ba_ref, Wb_ref, bb_ref,
                 mn_ref, out_ref, m_ref, den_ref, l2col_ref, xm2_ref):
    i = pl.program_id(0)
    nsteps = pl.num_programs(0)
    xb = xb_ref[...]

    @pl.when(i == 0)
    def _():
        x = x_ref[...]
        l2col_ref[...] = jnp.sum(x * x, axis=1, keepdims=True)
        xm2_ref[...] = (-2.0 * x).astype(jnp.bfloat16)

    l2b_row = _dot(jnp.ones((1, D), jnp.float32), xb * xb, ((1,), (1,)))
    dm2t = jax.lax.dot_general(
        xm2_ref[...], xb.astype(jnp.bfloat16),
        dimension_numbers=((((1,), (1,))), ((), ())),
        preferred_element_type=jnp.float32)
    u = l2col_ref[...] + dm2t
    colmin = jnp.min(u, axis=0, keepdims=True)
    mv = jnp.min(colmin + l2b_row).reshape(1, 1)

    @pl.when(i == 0)
    def _():
        mn_ref[...] = mv

    @pl.when(i != 0)
    def _():
        mn_ref[...] = jnp.minimum(mn_ref[...], mv)

    h = xb
    for W_ref, b_ref in ((W1_ref, b1_ref), (W2_ref, b2_ref), (W3_ref, b3_ref)):
        Wd = W_ref[0] - W_ref[2]
        h = jnp.maximum(_dot(h, Wd, ((1,), (0,))) + b_ref[...].reshape(1, -1), 0.0)
    t = jnp.tanh(_dot(h, Wa_ref[...], ((1,), (0,))) + ba_ref[...].reshape(1, -1))
    g = _dot(t, Wb_ref[...], ((1,), (0,))) + bb_ref[...].reshape(1, 1)

    row = jax.lax.broadcasted_iota(jnp.int32, (BM, SEG), 0)
    col = jax.lax.broadcasted_iota(jnp.int32, (BM, SEG), 1)
    mask = (row % SEG) == col
    ninf = jnp.float32(-jnp.inf)
    mb = jnp.max(jnp.where(mask, g, ninf), axis=0, keepdims=True)

    @pl.when(i == 0)
    def _():
        e = jnp.where(mask, jnp.exp(g - mb), 0.0)
        m_ref[...] = mb
        den_ref[...] = jnp.sum(e, axis=0, keepdims=True)
        out_ref[...] = _dot(e, h, ((0,), (0,)))

    @pl.when(i != 0)
    def _():
        m_old = m_ref[...]
        m_new = jnp.maximum(m_old, mb)
        scale = jnp.exp(m_old - m_new)
        e = jnp.where(mask, jnp.exp(g - m_new), 0.0)
        den_ref[...] = den_ref[...] * scale + jnp.sum(e, axis=0, keepdims=True)
        out_ref[...] = (_dot(_diag(scale), out_ref[...], ((1,), (0,)))
                        + _dot(e, h, ((0,), (0,))))
        m_ref[...] = m_new

    @pl.when(i == nsteps - 1)
    def _():
        out_ref[...] = _dot(_diag(1.0 / den_ref[...]), out_ref[...],
                            ((1,), (0,)))


def _mega(x, W1, b1, W2, b2, W3, b3, Wa, ba, Wb, bb):
    mn, out = pl.pallas_call(
        _mega_kernel,
        grid=(N // BM,),
        in_specs=[
            pl.BlockSpec((BM, D), lambda i: (i, 0)),
            pl.BlockSpec((N, D), lambda i: (0, 0)),
            pl.BlockSpec((3, D, D), lambda i: (0, 0, 0)),
            pl.BlockSpec((D,), lambda i: (0,)),
            pl.BlockSpec((3, D, D), lambda i: (0, 0, 0)),
            pl.BlockSpec((D,), lambda i: (0,)),
            pl.BlockSpec((3, D, D), lambda i: (0, 0, 0)),
            pl.BlockSpec((D,), lambda i: (0,)),
            pl.BlockSpec((D, ATT), lambda i: (0, 0)),
            pl.BlockSpec((ATT,), lambda i: (0,)),
            pl.BlockSpec((ATT, 1), lambda i: (0, 0)),
            pl.BlockSpec((1,), lambda i: (0,)),
        ],
        out_specs=[
            pl.BlockSpec((1, 1), lambda i: (0, 0)),
            pl.BlockSpec((SEG, D), lambda i: (0, 0)),
        ],
        out_shape=[
            jax.ShapeDtypeStruct((1, 1), jnp.float32),
            jax.ShapeDtypeStruct((SEG, D), jnp.float32),
        ],
        scratch_shapes=[
            pltpu.VMEM((1, SEG), jnp.float32),
            pltpu.VMEM((1, SEG), jnp.float32),
            pltpu.VMEM((N, 1), jnp.float32),
            pltpu.VMEM((N, D), jnp.bfloat16),
        ],
    )(x, x, W1, b1, W2, b2, W3, b3, Wa, ba, Wb, bb)
    return mn[0, 0], out


def _maxv_kernel(xb_ref, x_ref, maxv_ref):
    i = pl.program_id(0)
    m = jnp.max(_gram_block(xb_ref[...], x_ref[...])).reshape(1, 1)

    @pl.when(i == 0)
    def _():
        maxv_ref[...] = m

    @pl.when(i != 0)
    def _():
        maxv_ref[...] = jnp.maximum(maxv_ref[...], m)

def _adj_block(xb, x, thresh, i):
    v = _gram_block(xb, x)
    dist = jnp.sqrt(v)
    adj = dist < thresh
    gi = jax.lax.broadcasted_iota(jnp.int32, (BM, N), 0) + i * BM
    gj = jax.lax.broadcasted_iota(jnp.int32, (BM, N), 1)
    same = (gi % SEG) == (gj % SEG)
    return adj & jnp.logical_not(same)


def _deg_kernel(thresh_ref, xb_ref, x_ref, deg_ref):
    i = pl.program_id(0)
    adj = _adj_block(xb_ref[...], x_ref[...], thresh_ref[...], i)
    deg_ref[...] = jnp.sum(jnp.where(adj, 1.0, 0.0), axis=1, keepdims=True)


def _lw_kernel(thresh_ref, xb_ref, x_ref, deg_ref, lw_ref):
    i = pl.program_id(0)
    adj = _adj_block(xb_ref[...], x_ref[...], thresh_ref[...], i)
    deg = deg_ref[...]
    dinv = deg ** -0.5
    dinv = jnp.where(jnp.isinf(dinv), 0.0, dinv)
    deg_b = deg_ref[pl.ds(i * BM, BM), :]
    dinv_b = deg_b ** -0.5
    dinv_b = jnp.where(jnp.isinf(dinv_b), 0.0, dinv_b)
    lw_ref[...] = jnp.where(adj, (-dinv_b) * dinv.reshape(1, -1), 0.0)


def _lap_kernel(lw_ref, h_ref, out_ref):
    out_ref[...] = _dot(lw_ref[...], h_ref[...], ((1,), (0,)))


def _cheb_out_kernel(lw_ref, tx1_ref, h_ref, W_ref, b_ref, out_ref):
    i = pl.program_id(0)
    hb = h_ref[pl.ds(i * BM, BM), :]
    tx1b = tx1_ref[pl.ds(i * BM, BM), :]
    tx2b = 2.0 * _dot(lw_ref[...], tx1_ref[...], ((1,), (0,))) - hb
    acc = (_dot(hb, W_ref[0], ((1,), (0,)))
           + _dot(tx1b, W_ref[1], ((1,), (0,)))
           + _dot(tx2b, W_ref[2], ((1,), (0,)))
           + b_ref[...].reshape(1, -1))
    out_ref[...] = jnp.maximum(acc, 0.0)


def _pool_kernel(h_ref, Wa_ref, ba_ref, Wb_ref, bb_ref, out_ref):
    out_ref[...] = _attention_pool(h_ref[...], Wa_ref[...], ba_ref[...],
                                   Wb_ref[...], bb_ref[...])


def _full_graph_path(x, W1, b1, W2, b2, W3, b3, Wa, ba, Wb, bb):
    maxv = pl.pallas_call(
        _maxv_kernel,
        grid=(N // BM,),
        in_specs=[
            pl.BlockSpec((BM, D), lambda i: (i, 0)),
            pl.BlockSpec((N, D), lambda i: (0, 0)),
        ],
        out_specs=pl.BlockSpec((1, 1), lambda i: (0, 0)),
        out_shape=jax.ShapeDtypeStruct((1, 1), jnp.float32),
    )(x, x)
    thresh = 0.5 * jnp.sqrt(maxv)

    deg = pl.pallas_call(
        _deg_kernel,
        grid=(N // BM,),
        in_specs=[
            pl.BlockSpec((1, 1), lambda i: (0, 0)),
            pl.BlockSpec((BM, D), lambda i: (i, 0)),
            pl.BlockSpec((N, D), lambda i: (0, 0)),
        ],
        out_specs=pl.BlockSpec((BM, 1), lambda i: (i, 0)),
        out_shape=jax.ShapeDtypeStruct((N, 1), jnp.float32),
    )(thresh, x, x)

    lw = pl.pallas_call(
        _lw_kernel,
        grid=(N // BM,),
        in_specs=[
            pl.BlockSpec((1, 1), lambda i: (0, 0)),
            pl.BlockSpec((BM, D), lambda i: (i, 0)),
            pl.BlockSpec((N, D), lambda i: (0, 0)),
            pl.BlockSpec((N, 1), lambda i: (0, 0)),
        ],
        out_specs=pl.BlockSpec((BM, N), lambda i: (i, 0)),
        out_shape=jax.ShapeDtypeStruct((N, N), jnp.float32),
    )(thresh, x, x, deg)

    lap = pl.pallas_call(
        _lap_kernel,
        grid=(N // BM,),
        in_specs=[
            pl.BlockSpec((BM, N), lambda i: (i, 0)),
            pl.BlockSpec((N, D), lambda i: (0, 0)),
        ],
        out_specs=pl.BlockSpec((BM, D), lambda i: (i, 0)),
        out_shape=jax.ShapeDtypeStruct((N, D), jnp.float32),
    )

    cheb_out = pl.pallas_call(
        _cheb_out_kernel,
        grid=(N // BM,),
        in_specs=[
            pl.BlockSpec((BM, N), lambda i: (i, 0)),
            pl.BlockSpec((N, D), lambda i: (0, 0)),
            pl.BlockSpec((N, D), lambda i: (0, 0)),
            pl.BlockSpec((3, D, D), lambda i: (0, 0, 0)),
            pl.BlockSpec((D,), lambda i: (0,)),
        ],
        out_specs=pl.BlockSpec((BM, D), lambda i: (i, 0)),
        out_shape=jax.ShapeDtypeStruct((N, D), jnp.float32),
    )

    h = x
    for W, b in ((W1, b1), (W2, b2), (W3, b3)):
        tx1 = lap(lw, h)
        h = cheb_out(lw, tx1, h, W, b)

    return pl.pallas_call(
        _pool_kernel,
        out_shape=jax.ShapeDtypeStruct((SEG, D), jnp.float32),
    )(h, Wa, ba, Wb, bb)


def kernel(feats, W1, b1, W2, b2, W3, b3, Wa, ba, Wb, bb):
    bs, bag, d = feats.shape
    x = feats.reshape(bs * bag, d)
    mn, out_fast = _mega(x, W1, b1, W2, b2, W3, b3, Wa, ba, Wb, bb)
    out = jax.lax.cond(
        mn < 0,
        lambda ops: ops[0],
        lambda ops: _full_graph_path(*ops[1:]),
        (out_fast, x, W1, b1, W2, b2, W3, b3, Wa, ba, Wb, bb),
    )
    return out.reshape(bs, d)
